# baseline (speedup 1.0000x reference)
import functools

import jax
import jax.numpy as jnp
from jax import lax
from jax.experimental import pallas as pl
from jax.experimental.pallas import tpu as pltpu
from jax.experimental.pallas import tpu_sc as plsc

_INTERPRET = False

Q = 64
D = 32
K = 8
CHUNK = 32768
SEG = 512
SEGS = CHUNK // SEG

_NEG_INF = float("-inf")
_I32_MAX = 2**31 - 1


def _stage_a_body(q_ref, k_ref, s_ref, b_ref, *, m_total):
    i = pl.program_id(0)
    q = q_ref[...]
    k = k_ref[...]
    s = jax.lax.dot_general(
        q, k, dimension_numbers=(((1,), (1,)), ((), ())),
        preferred_element_type=jnp.float32)
    col = i * CHUNK + jax.lax.broadcasted_iota(jnp.int32, (Q, CHUNK), 1)
    s = jnp.where(col < m_total, s, _NEG_INF)
    s3 = s.reshape(Q, SEGS, SEG)
    s_ref[...] = jnp.transpose(s3, (1, 0, 2)).reshape(SEGS * Q, SEG)
    b_ref[0] = jnp.max(s3, axis=2)


def _stage_b_body(bm_ref, bid_ref, *, nb):
    bm = bm_ref[...]
    cols = jax.lax.broadcasted_iota(jnp.int32, (Q, nb), 1)
    ids = []
    for _ in range(K):
        m = jnp.max(bm, axis=1, keepdims=True)
        pick = jnp.min(jnp.where(bm == m, cols, _I32_MAX),
                       axis=1, keepdims=True)
        ids.append(pick)
        bm = jnp.where(cols == pick, _NEG_INF, bm)
    bid_ref[...] = jnp.concatenate(ids, axis=1)


_SC_NC = 2
_SC_NS = 16
_SC_NW = _SC_NC * _SC_NS
_GATHER_N = Q * K
_G_PER_W = _GATHER_N // _SC_NW


def _sc_gather_body(table_hbm, idx_hbm, out_hbm, idx_v, rows_v, sem):
    wid = lax.axis_index("s") * _SC_NC + lax.axis_index("c")
    base = wid * _G_PER_W
    pltpu.sync_copy(idx_hbm.at[pl.ds(base, _G_PER_W)], idx_v)
    pltpu.async_copy(table_hbm.at[idx_v], rows_v, sem).wait()
    pltpu.sync_copy(rows_v, out_hbm.at[pl.ds(base, _G_PER_W)])


def _stage_d_body(cand_ref, bid_ref, idx_ref):
    v = cand_ref[...]
    bid = bid_ref[...]
    lane = jax.lax.broadcasted_iota(jnp.int32, (Q, K, SEG), 2)
    gcol = bid[:, :, None] * SEG + lane
    picks = []
    for _ in range(K):
        m = jnp.max(v, axis=(1, 2), keepdims=True)
        pick = jnp.min(jnp.where(v == m, gcol, _I32_MAX),
                       axis=(1, 2), keepdims=True)
        picks.append(pick.reshape(Q, 1))
        v = jnp.where(gcol == pick, _NEG_INF, v)
    idx_ref[...] = jnp.concatenate(picks, axis=1)


def _stage_e_body(idx_s, *refs):
    (v0, v1, v2, v3, v4, v5, v6, v7, o_ref) = refs
    q = pl.program_id(0)
    sub = jax.lax.broadcasted_iota(jnp.int32, (8, 1), 0)
    rows = []
    for j, v in enumerate((v0, v1, v2, v3, v4, v5, v6, v7)):
        r = idx_s[q * K + j] % 8
        rows.append(jnp.sum(jnp.where(sub == r, v[...], 0.0),
                            axis=0, keepdims=True))
    o_ref[0] = jnp.concatenate(rows, axis=0)


def kernel(query, keys, values, n_per_key):
    m_total = keys.shape[0]
    nchunks = -(-m_total // CHUNK)
    nb = nchunks * SEGS

    sims2, bmax3 = pl.pallas_call(
        functools.partial(_stage_a_body, m_total=m_total),
        grid=(nchunks,),
        in_specs=[
            pl.BlockSpec((Q, D), lambda i: (0, 0)),
            pl.BlockSpec((CHUNK, D), lambda i: (i, 0)),
        ],
        out_specs=[
            pl.BlockSpec((SEGS * Q, SEG), lambda i: (i, 0)),
            pl.BlockSpec((1, Q, SEGS), lambda i: (i, 0, 0)),
        ],
        out_shape=[
            jax.ShapeDtypeStruct((nb * Q, SEG), jnp.float32),
            jax.ShapeDtypeStruct((nchunks, Q, SEGS), jnp.float32),
        ],
        compiler_params=pltpu.CompilerParams(
            dimension_semantics=("parallel",)),
        interpret=_INTERPRET,
    )(query, keys)

    bmax = jnp.transpose(bmax3, (1, 0, 2)).reshape(Q, nb)
    bid = pl.pallas_call(
        functools.partial(_stage_b_body, nb=nb),
        out_shape=jax.ShapeDtypeStruct((Q, K), jnp.int32),
        interpret=_INTERPRET,
    )(bmax)

    row_idx = (bid * Q + jnp.arange(Q, dtype=jnp.int32)[:, None]
               ).reshape(Q * K)
    sc_gather = functools.partial(
        pl.kernel,
        mesh=plsc.VectorSubcoreMesh(core_axis_name="c", subcore_axis_name="s"),
        out_type=jax.ShapeDtypeStruct((_GATHER_N, SEG), jnp.float32),
        scratch_types=[
            pltpu.VMEM((_G_PER_W,), jnp.int32),
            pltpu.VMEM((_G_PER_W, SEG), jnp.float32),
            pltpu.SemaphoreType.DMA,
        ],
    )(_sc_gather_body)
    cand = sc_gather(sims2, row_idx)

    idx2 = pl.pallas_call(
        _stage_d_body,
        out_shape=jax.ShapeDtypeStruct((Q, K), jnp.int32),
        interpret=_INTERPRET,
    )(cand.reshape(Q, K, SEG), bid)

    idx_flat = jnp.clip(idx2.reshape(Q * K) + (n_per_key - K),
                        0, m_total - 1).astype(jnp.int32)
    grid_e = pltpu.PrefetchScalarGridSpec(
        num_scalar_prefetch=1,
        grid=(Q,),
        in_specs=[
            pl.BlockSpec((8, D),
                         functools.partial(
                             lambda j, q, s: (s[q * K + j] // 8, 0), j))
            for j in range(K)
        ],
        out_specs=pl.BlockSpec((1, K, D), lambda q, s: (q, 0, 0)),
    )
    recalled = pl.pallas_call(
        _stage_e_body,
        grid_spec=grid_e,
        out_shape=jax.ShapeDtypeStruct((Q, K, D), jnp.float32),
        interpret=_INTERPRET,
    )(idx_flat, *([values] * K))

    return recalled


# speedup vs baseline: 1.0189x; 1.0189x over previous
import functools

import jax
import jax.numpy as jnp
from jax import lax
from jax.experimental import pallas as pl
from jax.experimental.pallas import tpu as pltpu
from jax.experimental.pallas import tpu_sc as plsc

_INTERPRET = False

Q = 64
D = 32
K = 8
CHUNK = 16384
SEG = 512
SEGS = CHUNK // SEG

_NEG_INF = float("-inf")
_I32_MAX = 2**31 - 1


def _stage_a_body(q_ref, k_ref, s_ref, b_ref, *, m_total):
    i = pl.program_id(0)
    q = q_ref[...]
    k = k_ref[...]
    s = jax.lax.dot_general(
        q, k, dimension_numbers=(((1,), (1,)), ((), ())),
        preferred_element_type=jnp.float32)
    col = i * CHUNK + jax.lax.broadcasted_iota(jnp.int32, (Q, CHUNK), 1)
    s = jnp.where(col < m_total, s, _NEG_INF)
    s3 = s.reshape(Q, SEGS, SEG)
    s_ref[...] = jnp.transpose(s3, (1, 0, 2)).reshape(SEGS * Q, SEG)
    b_ref[0] = jnp.max(s3, axis=2)


def _stage_b_body(bm_ref, bid_ref, *, nb):
    bm = bm_ref[...]
    cols = jax.lax.broadcasted_iota(jnp.int32, (Q, nb), 1)
    ids = []
    for _ in range(K):
        m = jnp.max(bm, axis=1, keepdims=True)
        pick = jnp.min(jnp.where(bm == m, cols, _I32_MAX),
                       axis=1, keepdims=True)
        ids.append(pick)
        bm = jnp.where(cols == pick, _NEG_INF, bm)
    bid_ref[...] = jnp.concatenate(ids, axis=1)


_SC_NC = 2
_SC_NS = 16
_SC_NW = _SC_NC * _SC_NS
_GATHER_N = Q * K
_G_PER_W = _GATHER_N // _SC_NW


def _sc_gather_body(table_hbm, idx_hbm, out_hbm, idx_v, rows_v, sem):
    wid = lax.axis_index("s") * _SC_NC + lax.axis_index("c")
    base = wid * _G_PER_W
    pltpu.sync_copy(idx_hbm.at[pl.ds(base, _G_PER_W)], idx_v)
    pltpu.async_copy(table_hbm.at[idx_v], rows_v, sem).wait()
    pltpu.sync_copy(rows_v, out_hbm.at[pl.ds(base, _G_PER_W)])


def _stage_d_body(cand_ref, bid_ref, idx_ref):
    v = cand_ref[...]
    bid = bid_ref[...]
    lane = jax.lax.broadcasted_iota(jnp.int32, (Q, K, SEG), 2)
    gcol = bid[:, :, None] * SEG + lane
    picks = []
    for _ in range(K):
        m = jnp.max(v, axis=(1, 2), keepdims=True)
        pick = jnp.min(jnp.where(v == m, gcol, _I32_MAX),
                       axis=(1, 2), keepdims=True)
        picks.append(pick.reshape(Q, 1))
        v = jnp.where(gcol == pick, _NEG_INF, v)
    idx_ref[...] = jnp.concatenate(picks, axis=1)


_E_QPS = 8
_E_OPS = _E_QPS * K


def _stage_e_body(idx_s, *refs):
    vrefs, o_ref = refs[:_E_OPS], refs[_E_OPS]
    g = pl.program_id(0)
    sub = jax.lax.broadcasted_iota(jnp.int32, (8, 1), 0)
    rows = []
    for o, v in enumerate(vrefs):
        r = idx_s[(g * _E_QPS + o // K) * K + (o % K)] % 8
        rows.append(jnp.sum(jnp.where(sub == r, v[...], 0.0),
                            axis=0, keepdims=True))
    o_ref[...] = jnp.concatenate(rows, axis=0).reshape(_E_QPS, K, D)


def kernel(query, keys, values, n_per_key):
    m_total = keys.shape[0]
    nchunks = -(-m_total // CHUNK)
    nb = nchunks * SEGS

    sims2, bmax3 = pl.pallas_call(
        functools.partial(_stage_a_body, m_total=m_total),
        grid=(nchunks,),
        in_specs=[
            pl.BlockSpec((Q, D), lambda i: (0, 0)),
            pl.BlockSpec((CHUNK, D), lambda i: (i, 0)),
        ],
        out_specs=[
            pl.BlockSpec((SEGS * Q, SEG), lambda i: (i, 0)),
            pl.BlockSpec((1, Q, SEGS), lambda i: (i, 0, 0)),
        ],
        out_shape=[
            jax.ShapeDtypeStruct((nb * Q, SEG), jnp.float32),
            jax.ShapeDtypeStruct((nchunks, Q, SEGS), jnp.float32),
        ],
        compiler_params=pltpu.CompilerParams(
            dimension_semantics=("parallel",)),
        interpret=_INTERPRET,
    )(query, keys)

    bmax = jnp.transpose(bmax3, (1, 0, 2)).reshape(Q, nb)
    bid = pl.pallas_call(
        functools.partial(_stage_b_body, nb=nb),
        out_shape=jax.ShapeDtypeStruct((Q, K), jnp.int32),
        interpret=_INTERPRET,
    )(bmax)

    row_idx = (bid * Q + jnp.arange(Q, dtype=jnp.int32)[:, None]
               ).reshape(Q * K)
    sc_gather = functools.partial(
        pl.kernel,
        mesh=plsc.VectorSubcoreMesh(core_axis_name="c", subcore_axis_name="s"),
        out_type=jax.ShapeDtypeStruct((_GATHER_N, SEG), jnp.float32),
        scratch_types=[
            pltpu.VMEM((_G_PER_W,), jnp.int32),
            pltpu.VMEM((_G_PER_W, SEG), jnp.float32),
            pltpu.SemaphoreType.DMA,
        ],
    )(_sc_gather_body)
    cand = sc_gather(sims2, row_idx)

    idx2 = pl.pallas_call(
        _stage_d_body,
        out_shape=jax.ShapeDtypeStruct((Q, K), jnp.int32),
        interpret=_INTERPRET,
    )(cand.reshape(Q, K, SEG), bid)

    idx_flat = jnp.clip(idx2.reshape(Q * K) + (n_per_key - K),
                        0, m_total - 1).astype(jnp.int32)
    grid_e = pltpu.PrefetchScalarGridSpec(
        num_scalar_prefetch=1,
        grid=(Q // _E_QPS,),
        in_specs=[
            pl.BlockSpec((8, D),
                         functools.partial(
                             lambda o, g, s: (
                                 s[(g * _E_QPS + o // K) * K + (o % K)] // 8,
                                 0), o))
            for o in range(_E_OPS)
        ],
        out_specs=pl.BlockSpec((_E_QPS, K, D), lambda g, s: (g, 0, 0)),
    )
    recalled = pl.pallas_call(
        _stage_e_body,
        grid_spec=grid_e,
        out_shape=jax.ShapeDtypeStruct((Q, K, D), jnp.float32),
        interpret=_INTERPRET,
    )(idx_flat, *([values] * _E_OPS))

    return recalled


# speedup vs baseline: 1.0402x; 1.0209x over previous
import functools

import jax
import jax.numpy as jnp
from jax import lax
from jax.experimental import pallas as pl
from jax.experimental.pallas import tpu as pltpu
from jax.experimental.pallas import tpu_sc as plsc

_INTERPRET = False

Q = 64
D = 32
K = 8
CHUNK = 16384
SEG = 512
SEGS = CHUNK // SEG

_NEG_INF = float("-inf")
_I32_MAX = 2**31 - 1


def _stage_a_body(q_ref, k_ref, s_ref, b_ref, *, m_total):
    i = pl.program_id(0)
    q = q_ref[...]
    k = k_ref[...]
    s = jax.lax.dot_general(
        q, k, dimension_numbers=(((1,), (1,)), ((), ())),
        preferred_element_type=jnp.float32)
    col = i * CHUNK + jax.lax.broadcasted_iota(jnp.int32, (Q, CHUNK), 1)
    s = jnp.where(col < m_total, s, _NEG_INF)
    s3 = s.reshape(Q, SEGS, SEG)
    s_ref[...] = jnp.transpose(s3, (1, 0, 2)).reshape(SEGS * Q, SEG)
    b_ref[0] = jnp.max(s3, axis=2)


def _stage_b_body(bm_ref, bid_ref, *, nb):
    bm = bm_ref[...]
    cols = jax.lax.broadcasted_iota(jnp.int32, (Q, nb), 1)
    ids = []
    for _ in range(K):
        m = jnp.max(bm, axis=1, keepdims=True)
        pick = jnp.min(jnp.where(bm == m, cols, _I32_MAX),
                       axis=1, keepdims=True)
        ids.append(pick)
        bm = jnp.where(cols == pick, _NEG_INF, bm)
    bid_ref[...] = jnp.concatenate(ids, axis=1)


_SC_NC = 2
_SC_NS = 16
_SC_NW = _SC_NC * _SC_NS
_GATHER_N = Q * K
_G_PER_W = _GATHER_N // _SC_NW


def _sc_gather_body(table_hbm, idx_hbm, out_hbm, idx_v, rows_v, sem):
    wid = lax.axis_index("s") * _SC_NC + lax.axis_index("c")
    base = wid * _G_PER_W
    pltpu.sync_copy(idx_hbm.at[pl.ds(base, _G_PER_W)], idx_v)
    pltpu.async_copy(table_hbm.at[idx_v], rows_v, sem).wait()
    pltpu.sync_copy(rows_v, out_hbm.at[pl.ds(base, _G_PER_W)])


def _stage_d_body(cand_ref, bid_ref, idx_ref):
    v = cand_ref[...]
    bid = bid_ref[...]
    lane = jax.lax.broadcasted_iota(jnp.int32, (Q, K, SEG), 2)
    gcol = bid[:, :, None] * SEG + lane
    picks = []
    for _ in range(K):
        m = jnp.max(v, axis=(1, 2), keepdims=True)
        pick = jnp.min(jnp.where(v == m, gcol, _I32_MAX),
                       axis=(1, 2), keepdims=True)
        picks.append(pick.reshape(Q, 1))
        v = jnp.where(gcol == pick, _NEG_INF, v)
    idx_ref[...] = jnp.concatenate(picks, axis=1)


def _stage_e_body(idx_ref, v_ref, o_ref, rows_v, sem):
    copies = []
    for i in range(_GATHER_N):
        cp = pltpu.make_async_copy(
            v_ref.at[pl.ds(idx_ref[i], 1), :],
            rows_v.at[pl.ds(i, 1), :],
            sem)
        cp.start()
        copies.append(cp)
    for cp in copies:
        cp.wait()
    o_ref[...] = rows_v[...].reshape(Q, K, D)


def kernel(query, keys, values, n_per_key):
    m_total = keys.shape[0]
    nchunks = -(-m_total // CHUNK)
    nb = nchunks * SEGS

    sims2, bmax3 = pl.pallas_call(
        functools.partial(_stage_a_body, m_total=m_total),
        grid=(nchunks,),
        in_specs=[
            pl.BlockSpec((Q, D), lambda i: (0, 0)),
            pl.BlockSpec((CHUNK, D), lambda i: (i, 0)),
        ],
        out_specs=[
            pl.BlockSpec((SEGS * Q, SEG), lambda i: (i, 0)),
            pl.BlockSpec((1, Q, SEGS), lambda i: (i, 0, 0)),
        ],
        out_shape=[
            jax.ShapeDtypeStruct((nb * Q, SEG), jnp.float32),
            jax.ShapeDtypeStruct((nchunks, Q, SEGS), jnp.float32),
        ],
        compiler_params=pltpu.CompilerParams(
            dimension_semantics=("parallel",)),
        interpret=_INTERPRET,
    )(query, keys)

    bmax = jnp.transpose(bmax3, (1, 0, 2)).reshape(Q, nb)
    bid = pl.pallas_call(
        functools.partial(_stage_b_body, nb=nb),
        out_shape=jax.ShapeDtypeStruct((Q, K), jnp.int32),
        interpret=_INTERPRET,
    )(bmax)

    row_idx = (bid * Q + jnp.arange(Q, dtype=jnp.int32)[:, None]
               ).reshape(Q * K)
    sc_gather = functools.partial(
        pl.kernel,
        mesh=plsc.VectorSubcoreMesh(core_axis_name="c", subcore_axis_name="s"),
        out_type=jax.ShapeDtypeStruct((_GATHER_N, SEG), jnp.float32),
        scratch_types=[
            pltpu.VMEM((_G_PER_W,), jnp.int32),
            pltpu.VMEM((_G_PER_W, SEG), jnp.float32),
            pltpu.SemaphoreType.DMA,
        ],
    )(_sc_gather_body)
    cand = sc_gather(sims2, row_idx)

    idx2 = pl.pallas_call(
        _stage_d_body,
        out_shape=jax.ShapeDtypeStruct((Q, K), jnp.int32),
        interpret=_INTERPRET,
    )(cand.reshape(Q, K, SEG), bid)

    idx_flat = jnp.clip(idx2.reshape(Q * K) + (n_per_key - K),
                        0, m_total - 1).astype(jnp.int32)
    recalled = pl.pallas_call(
        _stage_e_body,
        in_specs=[
            pl.BlockSpec(memory_space=pltpu.SMEM),
            pl.BlockSpec(memory_space=pl.ANY),
        ],
        out_specs=pl.BlockSpec(memory_space=pltpu.VMEM),
        out_shape=jax.ShapeDtypeStruct((Q, K, D), jnp.float32),
        scratch_shapes=[
            pltpu.VMEM((_GATHER_N, D), jnp.float32),
            pltpu.SemaphoreType.DMA,
        ],
        interpret=_INTERPRET,
    )(idx_flat, values)

    return recalled
